# baseline (speedup 1.0000x reference)
import functools

import jax
import jax.numpy as jnp
from jax import lax
from jax.experimental import pallas as pl
from jax.experimental.pallas import tpu as pltpu
from jax.experimental.pallas import tpu_sc as plsc

VOCAB = 100000
EMBED_DIM = 300
D_PAD = 304
BATCH = 4096
SEQ_LEN = 200

_B = BATCH * SEQ_LEN
_NW = 32
_B_PER_W = _B // _NW
_CHUNK = 128
_N_CHUNKS = _B_PER_W // _CHUNK


def _make_sc_gather():
    mesh = plsc.VectorSubcoreMesh(core_axis_name="c", subcore_axis_name="s")

    @functools.partial(
        pl.kernel,
        mesh=mesh,
        out_type=jax.ShapeDtypeStruct((_B, D_PAD), jnp.float32),
        compiler_params=pltpu.CompilerParams(use_tc_tiling_on_sc=False),
        scratch_types=[
            pltpu.VMEM((_N_CHUNKS, _CHUNK), jnp.int32),
            pltpu.VMEM((_CHUNK, D_PAD), jnp.float32),
            pltpu.VMEM((_CHUNK, D_PAD), jnp.float32),
            pltpu.SemaphoreType.DMA,
            pltpu.SemaphoreType.DMA,
        ],
    )
    def k(idx_hbm, table_hbm, out_hbm, idx_v, buf0, buf1, sem0, sem1):
        wid = lax.axis_index("s") * 2 + lax.axis_index("c")
        base = wid * _B_PER_W

        pltpu.sync_copy(idx_hbm.at[wid], idx_v)

        bufs = (buf0, buf1)
        sems = (sem0, sem1)

        pltpu.async_copy(table_hbm.at[idx_v.at[0]], buf0, sem0)

        def outer(i, carry):
            for p in range(2):
                j = i * 2 + p
                cur, cur_sem = bufs[p], sems[p]
                nxt, nxt_sem = bufs[1 - p], sems[1 - p]

                @pl.when(j + 1 < _N_CHUNKS)
                def _():
                    pltpu.async_copy(table_hbm.at[idx_v.at[j + 1]], nxt, nxt_sem)

                pltpu.make_async_copy(table_hbm.at[idx_v.at[j]], cur, cur_sem).wait()
                pltpu.sync_copy(cur, out_hbm.at[pl.ds(base + j * _CHUNK, _CHUNK)])
            return carry

        lax.fori_loop(0, _N_CHUNKS // 2, outer, 0)

    return k


_sc_gather = _make_sc_gather()


def kernel(indices, table):
    idx = indices.reshape(_NW, _N_CHUNKS, _CHUNK).astype(jnp.int32)
    table_pad = jnp.pad(table, ((0, 0), (0, D_PAD - EMBED_DIM)))
    out = _sc_gather(idx, table_pad)
    return out[:, :EMBED_DIM].reshape(BATCH, SEQ_LEN, EMBED_DIM)
